# Initial kernel scaffold; baseline (speedup 1.0000x reference)
import jax
import jax.numpy as jnp
from jax.experimental import pallas as pl


def kernel(pert_idx, G_go, G_go_weight, pert_emb_w, sg_W, sg_b, lin1_W, lin1_b, bn1_g, bn1_b, lin2_W, lin2_b, bn2_g, bn2_b):
    raise NotImplementedError("write your pallas kernel here")


# speedup vs baseline: 12.9791x; 12.9791x over previous
import functools

import jax
import jax.numpy as jnp
from jax import lax
from jax.experimental import pallas as pl
from jax.experimental.pallas import tpu as pltpu
from jax.experimental.pallas import tpu_sc as plsc

N = 10000
NPAD = 10240
H = 128
E = 320000
B = 4096
NC, NS = 2, 16
NW = NC * NS
W = 80
EROWS = E // W
TROWS = EROWS // NW
RPT = NPAD // NS


def _mesh():
    return plsc.VectorSubcoreMesh(core_axis_name="c", subcore_axis_name="s")


_SC_PARAMS = pltpu.CompilerParams(needs_layout_passes=False)
_SC_PARAMS_NT = pltpu.CompilerParams(needs_layout_passes=False,
                                     use_tc_tiling_on_sc=False)


def _deg_body(col_hbm, w_hbm, out_hbm, col_v, w_v, tmp_v, deg_sh):
    c = lax.axis_index("c")
    s = lax.axis_index("s")
    wid = c * NS + s
    zero16 = jnp.zeros((16,), jnp.float32)
    def zloop(i, _):
        tmp_v[pl.ds(i * 16, 16)] = zero16
        return 0
    lax.fori_loop(0, RPT // 16, zloop, 0)
    pltpu.sync_copy(tmp_v, deg_sh.at[pl.ds(s * RPT, RPT)])
    plsc.subcore_barrier()
    pltpu.sync_copy(col_hbm.at[wid], col_v)
    pltpu.sync_copy(w_hbm.at[wid], w_v)
    def body(j, _):
        pltpu.sync_copy(w_v.at[j], deg_sh.at[col_v.at[j]], add=True)
        return 0
    lax.fori_loop(0, TROWS, body, 0)
    plsc.subcore_barrier()
    pltpu.sync_copy(deg_sh.at[pl.ds(s * RPT, RPT)], tmp_v)
    pltpu.sync_copy(tmp_v, out_hbm.at[pl.ds(c * NPAD + s * RPT, RPT)])


def _deg_call(col2, w2):
    return pl.kernel(
        _deg_body,
        out_type=jax.ShapeDtypeStruct((NC * NPAD,), jnp.float32),
        mesh=_mesh(),
        compiler_params=_SC_PARAMS,
        scratch_types=[
            pltpu.VMEM((TROWS, W), jnp.int32),
            pltpu.VMEM((TROWS, W), jnp.float32),
            pltpu.VMEM((RPT,), jnp.float32),
            pltpu.VMEM_SHARED((NPAD,), jnp.float32),
        ],
    )(col2, w2)


HH = H // NC
TROWS2 = EROWS // NS


def _msg_body(rowp1_hbm, col_hbm, w_hbm, dinv_hbm, table_hbm, out_hbm,
              rowp1_v, col_v, w_v, dinv_v, gath_v, norm_v, acc_sh, sem):
    c = lax.axis_index("c")
    s = lax.axis_index("s")
    zero16 = jnp.zeros((16,), jnp.float32)
    def zrow(i, _):
        for f in range(HH // 16):
            gath_v[i, pl.ds(f * 16, 16)] = zero16
        return 0
    lax.fori_loop(0, W, zrow, 0)
    def zcopy(k, _):
        pltpu.sync_copy(gath_v, acc_sh.at[pl.ds(s * RPT + k * W, W)])
        return 0
    lax.fori_loop(0, RPT // W, zcopy, 0)
    pltpu.sync_copy(dinv_hbm, dinv_v)
    pltpu.sync_copy(rowp1_hbm.at[s], rowp1_v)
    pltpu.sync_copy(col_hbm.at[s], col_v)
    pltpu.sync_copy(w_hbm.at[s], w_v)
    plsc.subcore_barrier()

    def window(j, _):
        cp = pltpu.async_copy(table_hbm.at[c].at[rowp1_v.at[j]], gath_v, sem)
        for k in range(W // 16):
            sl = pl.ds(k * 16, 16)
            rv = rowp1_v[j, sl]
            cv = col_v[j, sl]
            wv = w_v[j, sl]
            dvr = plsc.load_gather(dinv_v, [rv - 1])
            dvc = plsc.load_gather(dinv_v, [cv])
            norm_v[sl] = dvr * wv * dvc
        cp.wait()
        def scale(e, _):
            splat = plsc.load_gather(norm_v, [jnp.zeros((16,), jnp.int32) + e])
            for f in range(HH // 16):
                sl = pl.ds(f * 16, 16)
                gath_v[e, sl] = gath_v[e, sl] * splat
            return 0
        lax.fori_loop(0, W, scale, 0)
        pltpu.sync_copy(gath_v, acc_sh.at[col_v.at[j]], add=True)
        return 0

    lax.fori_loop(0, TROWS2, window, 0)
    plsc.subcore_barrier()
    def wout(k, _):
        base = s * RPT + k * W
        pltpu.sync_copy(acc_sh.at[pl.ds(base, W)], gath_v)
        pltpu.sync_copy(gath_v, out_hbm.at[c, pl.ds(base, W)])
        return 0
    lax.fori_loop(0, RPT // W, wout, 0)


def _msg_call(rowp1, col2, w2, dinv, table2):
    return pl.kernel(
        _msg_body,
        out_type=jax.ShapeDtypeStruct((NC, NPAD, HH), jnp.float32),
        mesh=_mesh(),
        compiler_params=_SC_PARAMS_NT,
        scratch_types=[
            pltpu.VMEM((TROWS2, W), jnp.int32),
            pltpu.VMEM((TROWS2, W), jnp.int32),
            pltpu.VMEM((TROWS2, W), jnp.float32),
            pltpu.VMEM((NPAD,), jnp.float32),
            pltpu.VMEM((W, HH), jnp.float32),
            pltpu.VMEM((W,), jnp.float32),
            pltpu.VMEM_SHARED((NPAD, HH), jnp.float32),
            pltpu.SemaphoreType.DMA,
        ],
    )(rowp1, col2, w2, dinv, table2)


def _comb_body(a0, a1, xin, d2, o):
    o[...] = jnp.concatenate([a0[...], a1[...]], axis=1) + d2[...] * xin[...]


def _comb_call(acc, xin_pad, d2):
    blk = NPAD // 8
    return pl.pallas_call(
        _comb_body,
        grid=(8,),
        in_specs=[
            pl.BlockSpec((blk, HH), lambda i: (i, 0)),
            pl.BlockSpec((blk, HH), lambda i: (i, 0)),
            pl.BlockSpec((blk, H), lambda i: (i, 0)),
            pl.BlockSpec((blk, 1), lambda i: (i, 0)),
        ],
        out_specs=pl.BlockSpec((blk, H), lambda i: (i, 0)),
        out_shape=jax.ShapeDtypeStruct((NPAD, H), jnp.float32),
    )(acc[0], acc[1], xin_pad, d2)


def _bg_body(xout_hbm, idx_hbm, out_hbm, i0_v, i1_v, g0_v, g1_v, emb_v, sem):
    c = lax.axis_index("c")
    s = lax.axis_index("s")
    wid = c * NS + s
    bw = B // NW
    pltpu.sync_copy(idx_hbm.at[pl.ds(wid * bw, bw)], i0_v)
    pltpu.sync_copy(idx_hbm.at[pl.ds(NW * bw + wid * bw, bw)], i1_v)
    pltpu.async_copy(xout_hbm.at[i0_v], g0_v, sem).wait()
    pltpu.async_copy(xout_hbm.at[i1_v], g1_v, sem).wait()
    def body(k, _):
        for f in range(H // 16):
            sl = pl.ds(f * 16, 16)
            emb_v[k, sl] = g0_v[k, sl] + g1_v[k, sl]
        return 0
    lax.fori_loop(0, B // NW, body, 0)
    pltpu.sync_copy(emb_v, out_hbm.at[pl.ds(wid * (B // NW), B // NW)])


def _bg_call(xout, idx2):
    bw = B // NW
    return pl.kernel(
        _bg_body,
        out_type=jax.ShapeDtypeStruct((B, H), jnp.float32),
        mesh=_mesh(),
        compiler_params=_SC_PARAMS,
        scratch_types=[
            pltpu.VMEM((bw,), jnp.int32),
            pltpu.VMEM((bw,), jnp.int32),
            pltpu.VMEM((bw, H), jnp.float32),
            pltpu.VMEM((bw, H), jnp.float32),
            pltpu.VMEM((bw, H), jnp.float32),
            pltpu.SemaphoreType.DMA,
        ],
    )(xout, idx2)


def _mlp_body(emb, sgWt, sgb2, W1t, b1, g1, bb1, W2t, b2, g2, bb2, o):
    f32 = jnp.float32
    x = jnp.dot(emb[...], sgWt[...], preferred_element_type=f32,
                precision=lax.Precision.HIGHEST) + sgb2[...]
    h = jnp.dot(x, W1t[...], preferred_element_type=f32,
                precision=lax.Precision.HIGHEST) + b1[...]
    m = jnp.mean(h, axis=0, keepdims=True)
    v = jnp.mean((h - m) * (h - m), axis=0, keepdims=True)
    h = (h - m) * lax.rsqrt(v + 1e-5) * g1[...] + bb1[...]
    h = jnp.maximum(h, 0.0)
    h2 = jnp.dot(h, W2t[...], preferred_element_type=f32,
                 precision=lax.Precision.HIGHEST) + b2[...]
    m2 = jnp.mean(h2, axis=0, keepdims=True)
    v2 = jnp.mean((h2 - m2) * (h2 - m2), axis=0, keepdims=True)
    o[...] = (h2 - m2) * lax.rsqrt(v2 + 1e-5) * g2[...] + bb2[...]


def _mlp_call(emb, sg_W, sg_b, lin1_W, lin1_b, bn1_g, bn1_b,
              lin2_W, lin2_b, bn2_g, bn2_b):
    r = lambda a: a.reshape(1, -1)
    return pl.pallas_call(
        _mlp_body,
        out_shape=jax.ShapeDtypeStruct((B, H), jnp.float32),
    )(emb, sg_W.T, r(2.0 * sg_b), lin1_W.T, r(lin1_b), r(bn1_g), r(bn1_b),
      lin2_W.T, r(lin2_b), r(bn2_g), r(bn2_b))


def kernel(pert_idx, G_go, G_go_weight, pert_emb_w, sg_W, sg_b,
           lin1_W, lin1_b, bn1_g, bn1_b, lin2_W, lin2_b, bn2_g, bn2_b):
    row = G_go[0].astype(jnp.int32)
    col = G_go[1].astype(jnp.int32)
    rowp1 = (row + 1).reshape(NW, TROWS, W)
    col2 = col.reshape(NW, TROWS, W)
    w2 = G_go_weight.reshape(NW, TROWS, W)

    degp = _deg_call(col2, w2).reshape(NC, NPAD)
    dinv = lax.rsqrt(1.0 + degp[0] + degp[1])

    table2 = jnp.stack([pert_emb_w[:, :HH], pert_emb_w[:, HH:]])
    acc = _msg_call(rowp1.reshape(NS, TROWS2, W), col2.reshape(NS, TROWS2, W),
                    w2.reshape(NS, TROWS2, W), dinv, table2)

    xin_pad = jnp.pad(pert_emb_w[1:], ((0, NPAD - N), (0, 0)))
    d2 = (dinv * dinv).reshape(NPAD, 1)
    xout = _comb_call(acc, xin_pad, d2)

    idx2 = pert_idx.astype(jnp.int32).T.reshape(2 * NW * (B // NW))
    emb = _bg_call(xout, idx2)

    return _mlp_call(emb, sg_W, sg_b, lin1_W, lin1_b, bn1_g, bn1_b,
                     lin2_W, lin2_b, bn2_g, bn2_b)


# speedup vs baseline: 23.5641x; 1.8155x over previous
import jax
import jax.numpy as jnp
from jax import lax
from jax.experimental import pallas as pl
from jax.experimental.pallas import tpu as pltpu
from jax.experimental.pallas import tpu_sc as plsc

N = 10000
NPAD = 10240
H = 128
E = 320000
B = 4096
NC, NS = 2, 16
NW = NC * NS
W = 128
E_PAD = 327680
TROWS = E_PAD // NW // W
NWIN = E_PAD // NS // W
RPT = NPAD // NS
HH = H // NC
NBUF = 4


def _mesh():
    return plsc.VectorSubcoreMesh(core_axis_name="c", subcore_axis_name="s")


_SC_PARAMS = pltpu.CompilerParams(needs_layout_passes=False)
_SC_PARAMS_NT = pltpu.CompilerParams(needs_layout_passes=False,
                                     use_tc_tiling_on_sc=False)


def _deg_body(col_hbm, w_hbm, out_hbm, col_v, w_v, tmp_v, deg_sh):
    c = lax.axis_index("c")
    s = lax.axis_index("s")
    wid = c * NS + s
    zero16 = jnp.zeros((16,), jnp.float32)
    def zloop(i, _):
        tmp_v[pl.ds(i * 16, 16)] = zero16
        return 0
    lax.fori_loop(0, RPT // 16, zloop, 0)
    pltpu.sync_copy(tmp_v, deg_sh.at[pl.ds(s * RPT, RPT)])
    plsc.subcore_barrier()
    pltpu.sync_copy(col_hbm.at[wid], col_v)
    pltpu.sync_copy(w_hbm.at[wid], w_v)
    def body(j, _):
        pltpu.sync_copy(w_v.at[j], deg_sh.at[col_v.at[j]], add=True)
        return 0
    lax.fori_loop(0, TROWS, body, 0)
    plsc.subcore_barrier()
    pltpu.sync_copy(deg_sh.at[pl.ds(s * RPT, RPT)], tmp_v)
    pltpu.sync_copy(tmp_v, out_hbm.at[pl.ds(c * NPAD + s * RPT, RPT)])


def _deg_call(col2, w2):
    return pl.kernel(
        _deg_body,
        out_type=jax.ShapeDtypeStruct((NC * NPAD,), jnp.float32),
        mesh=_mesh(),
        compiler_params=_SC_PARAMS,
        scratch_types=[
            pltpu.VMEM((TROWS, W), jnp.int32),
            pltpu.VMEM((TROWS, W), jnp.float32),
            pltpu.VMEM((RPT,), jnp.float32),
            pltpu.VMEM_SHARED((NPAD,), jnp.float32),
        ],
    )(col2, w2)


CHUNK = 80
NCHUNK = NWIN // CHUNK


def _msg_body(rowp1_hbm, col_hbm, w_hbm, dinv_hbm, table_hbm, out_hbm,
              rowp1_v, col_v, w_v, dinv_v, g4_v, norm_v, acc_sh, gsem, ssem):
    c = lax.axis_index("c")
    s = lax.axis_index("s")
    zero16 = jnp.zeros((16,), jnp.float32)

    def zrow(i, _):
        for f in range(HH // 16):
            g4_v[0, i, pl.ds(f * 16, 16)] = zero16
        return 0
    lax.fori_loop(0, W, zrow, 0)
    def zcopy(k, _):
        pltpu.sync_copy(g4_v.at[0], acc_sh.at[pl.ds(s * RPT + k * W, W)])
        return 0
    lax.fori_loop(0, RPT // W, zcopy, 0)
    pltpu.sync_copy(dinv_hbm, dinv_v)
    plsc.subcore_barrier()

    def gath_desc(j, b):
        return pltpu.make_async_copy(table_hbm.at[c].at[rowp1_v.at[j]],
                                     g4_v.at[b], gsem)

    def scat_desc(j, b):
        return pltpu.make_async_copy(g4_v.at[b], acc_sh.at[col_v.at[j]], ssem)

    def chunk_body(ci, _):
        base = ci * CHUNK
        pltpu.sync_copy(rowp1_hbm.at[s].at[pl.ds(base, CHUNK)], rowp1_v)
        pltpu.sync_copy(col_hbm.at[s].at[pl.ds(base, CHUNK)], col_v)
        pltpu.sync_copy(w_hbm.at[s].at[pl.ds(base, CHUNK)], w_v)
        for b in range(NBUF - 1):
            gath_desc(b, b).start()

        def outer(i, _):
            for b in range(NBUF):
                j = i * NBUF + b
                for k in range(W // 16):
                    sl = pl.ds(k * 16, 16)
                    rv = rowp1_v[j, sl]
                    cv = col_v[j, sl]
                    wv = w_v[j, sl]
                    dvr = plsc.load_gather(dinv_v, [rv - 1])
                    dvc = plsc.load_gather(dinv_v, [cv])
                    norm_v[sl] = dvr * wv * dvc
                gath_desc(j, b).wait()
                def scale(e, _):
                    splat = plsc.load_gather(norm_v,
                                             [jnp.zeros((16,), jnp.int32) + e])
                    for f in range(HH // 16):
                        sl = pl.ds(f * 16, 16)
                        g4_v[b, e, sl] = g4_v[b, e, sl] * splat
                    return 0
                lax.fori_loop(0, W, scale, 0)
                scat_desc(j, b).start(add=True)
                bn = (b + NBUF - 1) % NBUF
                @pl.when(j >= 1)
                def _():
                    scat_desc(j - 1, bn).wait()
                @pl.when(j + NBUF - 1 < CHUNK)
                def _():
                    gath_desc(j + NBUF - 1, bn).start()
            return 0

        lax.fori_loop(0, CHUNK // NBUF, outer, 0)
        scat_desc(CHUNK - 1, (CHUNK - 1) % NBUF).wait()
        return 0

    lax.fori_loop(0, NCHUNK, chunk_body, 0)
    plsc.subcore_barrier()
    def wout(k, _):
        base = s * RPT + k * W
        pltpu.sync_copy(acc_sh.at[pl.ds(base, W)], g4_v.at[0])
        pltpu.sync_copy(g4_v.at[0], out_hbm.at[c, pl.ds(base, W)])
        return 0
    lax.fori_loop(0, RPT // W, wout, 0)


def _msg_call(rowp1, col2, w2, dinv, table2):
    return pl.kernel(
        _msg_body,
        out_type=jax.ShapeDtypeStruct((NC, NPAD, HH), jnp.float32),
        mesh=_mesh(),
        compiler_params=_SC_PARAMS_NT,
        scratch_types=[
            pltpu.VMEM((CHUNK, W), jnp.int32),
            pltpu.VMEM((CHUNK, W), jnp.int32),
            pltpu.VMEM((CHUNK, W), jnp.float32),
            pltpu.VMEM((NPAD,), jnp.float32),
            pltpu.VMEM((NBUF, W, HH), jnp.float32),
            pltpu.VMEM((W,), jnp.float32),
            pltpu.VMEM_SHARED((NPAD, HH), jnp.float32),
            pltpu.SemaphoreType.DMA,
            pltpu.SemaphoreType.DMA,
        ],
    )(rowp1, col2, w2, dinv, table2)


def _comb_body(a0, a1, xin, d2, o):
    o[...] = jnp.concatenate([a0[...], a1[...]], axis=1) + d2[...] * xin[...]


def _comb_call(acc, xin_pad, d2):
    blk = NPAD // 8
    return pl.pallas_call(
        _comb_body,
        grid=(8,),
        in_specs=[
            pl.BlockSpec((blk, HH), lambda i: (i, 0)),
            pl.BlockSpec((blk, HH), lambda i: (i, 0)),
            pl.BlockSpec((blk, H), lambda i: (i, 0)),
            pl.BlockSpec((blk, 1), lambda i: (i, 0)),
        ],
        out_specs=pl.BlockSpec((blk, H), lambda i: (i, 0)),
        out_shape=jax.ShapeDtypeStruct((NPAD, H), jnp.float32),
    )(acc[0], acc[1], xin_pad, d2)


def _bg_body(xout_hbm, idx_hbm, out_hbm, i0_v, i1_v, g0_v, g1_v, emb_v, sem):
    c = lax.axis_index("c")
    s = lax.axis_index("s")
    wid = c * NS + s
    bw = B // NW
    pltpu.sync_copy(idx_hbm.at[pl.ds(wid * bw, bw)], i0_v)
    pltpu.sync_copy(idx_hbm.at[pl.ds(NW * bw + wid * bw, bw)], i1_v)
    pltpu.async_copy(xout_hbm.at[i0_v], g0_v, sem).wait()
    pltpu.async_copy(xout_hbm.at[i1_v], g1_v, sem).wait()
    def body(k, _):
        for f in range(H // 16):
            sl = pl.ds(f * 16, 16)
            emb_v[k, sl] = g0_v[k, sl] + g1_v[k, sl]
        return 0
    lax.fori_loop(0, B // NW, body, 0)
    pltpu.sync_copy(emb_v, out_hbm.at[pl.ds(wid * (B // NW), B // NW)])


def _bg_call(xout, idx2):
    bw = B // NW
    return pl.kernel(
        _bg_body,
        out_type=jax.ShapeDtypeStruct((B, H), jnp.float32),
        mesh=_mesh(),
        compiler_params=_SC_PARAMS,
        scratch_types=[
            pltpu.VMEM((bw,), jnp.int32),
            pltpu.VMEM((bw,), jnp.int32),
            pltpu.VMEM((bw, H), jnp.float32),
            pltpu.VMEM((bw, H), jnp.float32),
            pltpu.VMEM((bw, H), jnp.float32),
            pltpu.SemaphoreType.DMA,
        ],
    )(xout, idx2)


def _mlp_body(emb, sgWt, sgb2, W1t, b1, g1, bb1, W2t, b2, g2, bb2, o):
    f32 = jnp.float32
    x = jnp.dot(emb[...], sgWt[...], preferred_element_type=f32,
                precision=lax.Precision.HIGHEST) + sgb2[...]
    h = jnp.dot(x, W1t[...], preferred_element_type=f32,
                precision=lax.Precision.HIGHEST) + b1[...]
    m = jnp.mean(h, axis=0, keepdims=True)
    v = jnp.mean((h - m) * (h - m), axis=0, keepdims=True)
    h = (h - m) * lax.rsqrt(v + 1e-5) * g1[...] + bb1[...]
    h = jnp.maximum(h, 0.0)
    h2 = jnp.dot(h, W2t[...], preferred_element_type=f32,
                 precision=lax.Precision.HIGHEST) + b2[...]
    m2 = jnp.mean(h2, axis=0, keepdims=True)
    v2 = jnp.mean((h2 - m2) * (h2 - m2), axis=0, keepdims=True)
    o[...] = (h2 - m2) * lax.rsqrt(v2 + 1e-5) * g2[...] + bb2[...]


def _mlp_call(emb, sg_W, sg_b, lin1_W, lin1_b, bn1_g, bn1_b,
              lin2_W, lin2_b, bn2_g, bn2_b):
    r = lambda a: a.reshape(1, -1)
    return pl.pallas_call(
        _mlp_body,
        out_shape=jax.ShapeDtypeStruct((B, H), jnp.float32),
    )(emb, sg_W.T, r(2.0 * sg_b), lin1_W.T, r(lin1_b), r(bn1_g), r(bn1_b),
      lin2_W.T, r(lin2_b), r(bn2_g), r(bn2_b))


def kernel(pert_idx, G_go, G_go_weight, pert_emb_w, sg_W, sg_b,
           lin1_W, lin1_b, bn1_g, bn1_b, lin2_W, lin2_b, bn2_g, bn2_b):
    row = G_go[0].astype(jnp.int32)
    col = G_go[1].astype(jnp.int32)
    npad_e = E_PAD - E
    spread = (jnp.arange(npad_e, dtype=jnp.int32) * 37) % N
    rowp1 = jnp.concatenate([row + 1, spread + 1]).reshape(NS, NWIN, W)
    col_p = jnp.concatenate([col, spread])
    w_p = jnp.concatenate([G_go_weight, jnp.zeros((npad_e,), jnp.float32)])
    col2 = col_p.reshape(NS, NWIN, W)
    w2 = w_p.reshape(NS, NWIN, W)

    degp = _deg_call(col2.reshape(NW, TROWS, W),
                     w2.reshape(NW, TROWS, W)).reshape(NC, NPAD)
    dinv = lax.rsqrt(1.0 + degp[0] + degp[1])

    table2 = jnp.stack([pert_emb_w[:, :HH], pert_emb_w[:, HH:]])
    acc = _msg_call(rowp1, col2, w2, dinv, table2)

    xin_pad = jnp.pad(pert_emb_w[1:], ((0, NPAD - N), (0, 0)))
    d2 = (dinv * dinv).reshape(NPAD, 1)
    xout = _comb_call(acc, xin_pad, d2)

    idx2 = pert_idx.astype(jnp.int32).T.reshape(2 * NW * (B // NW))
    emb = _bg_call(xout, idx2)

    return _mlp_call(emb, sg_W, sg_b, lin1_W, lin1_b, bn1_g, bn1_b,
                     lin2_W, lin2_b, bn2_g, bn2_b)
